# baseline (speedup 1.0000x reference)
import functools

import jax
import jax.numpy as jnp
from jax import lax
from jax.experimental import pallas as pl
from jax.experimental.pallas import tpu as pltpu
from jax.experimental.pallas import tpu_sc as plsc

BATCH = 16384
EMBED_DIM = 32
CHUNK = 128


def _sc_gather(item_id3, org_id3, item_table, org_table, n_workers, n_chunks):
    b_per_w = n_chunks * CHUNK
    mesh = plsc.VectorSubcoreMesh(core_axis_name="c", subcore_axis_name="s")

    @functools.partial(
        pl.kernel,
        out_type=(
            jax.ShapeDtypeStruct((BATCH, EMBED_DIM), jnp.float32),
            jax.ShapeDtypeStruct((BATCH, EMBED_DIM), jnp.float32),
        ),
        mesh=mesh,
        scratch_types=[
            pltpu.VMEM((n_chunks, CHUNK), jnp.int32),
            pltpu.VMEM((n_chunks, CHUNK), jnp.int32),
            pltpu.VMEM((b_per_w, EMBED_DIM), jnp.float32),
            pltpu.VMEM((b_per_w, EMBED_DIM), jnp.float32),
            pltpu.SemaphoreType.DMA,
        ],
        compiler_params=pltpu.CompilerParams(use_tc_tiling_on_sc=False),
    )
    def k(iid_hbm, oid_hbm, itab_hbm, otab_hbm, iout_hbm, oout_hbm,
          iidx_v, oidx_v, irows_v, orows_v, sem):
        wid = lax.axis_index("s") * 2 + lax.axis_index("c")
        base = wid * b_per_w
        pltpu.sync_copy(iid_hbm.at[wid], iidx_v)
        pltpu.sync_copy(oid_hbm.at[wid], oidx_v)
        copies = []
        for j in range(n_chunks):
            copies.append(pltpu.async_copy(
                itab_hbm.at[iidx_v.at[j]],
                irows_v.at[pl.ds(j * CHUNK, CHUNK)], sem))
            copies.append(pltpu.async_copy(
                otab_hbm.at[oidx_v.at[j]],
                orows_v.at[pl.ds(j * CHUNK, CHUNK)], sem))
        for c in copies:
            c.wait()
        pltpu.sync_copy(irows_v, iout_hbm.at[pl.ds(base, b_per_w)])
        pltpu.sync_copy(orows_v, oout_hbm.at[pl.ds(base, b_per_w)])

    return k(item_id3, org_id3, item_table, org_table)


def _mlp_body(ig_ref, og_ref, w1_ref, b1_ref, w2_ref, b2_ref, w3_ref, b3_ref,
              out_ref):
    c = jnp.concatenate([ig_ref[...], og_ref[...]], axis=-1)
    x = jnp.maximum(
        jnp.dot(c, w1_ref[...], preferred_element_type=jnp.float32)
        + b1_ref[...], 0.0)
    x = jnp.maximum(
        jnp.dot(x, w2_ref[...], preferred_element_type=jnp.float32)
        + b2_ref[...], 0.0)
    y = jnp.dot(x, w3_ref[...], preferred_element_type=jnp.float32) + b3_ref[...]
    out_ref[...] = jax.nn.sigmoid(y)


def _tc_mlp(ig, og, W1, b1, W2, b2, W3, b3, block_b=2048):
    n_blocks = BATCH // block_b
    full = lambda shape: pl.BlockSpec(shape, lambda i: (0, 0))
    return pl.pallas_call(
        _mlp_body,
        grid=(n_blocks,),
        in_specs=[
            pl.BlockSpec((block_b, EMBED_DIM), lambda i: (i, 0)),
            pl.BlockSpec((block_b, EMBED_DIM), lambda i: (i, 0)),
            full((2 * EMBED_DIM, 128)),
            full((1, 128)),
            full((128, 64)),
            full((1, 64)),
            full((64, 1)),
            full((1, 1)),
        ],
        out_specs=pl.BlockSpec((block_b, 1), lambda i: (i, 0)),
        out_shape=jax.ShapeDtypeStruct((BATCH, 1), jnp.float32),
    )(ig, og, W1, b1.reshape(1, -1), W2, b2.reshape(1, -1), W3,
      b3.reshape(1, -1))


def kernel(item_id, org_id, item_table, org_table, W1, b1, W2, b2, W3, b3):
    info = plsc.get_sparse_core_info()
    n_workers = info.num_cores * info.num_subcores
    n_chunks = BATCH // (n_workers * CHUNK)
    item_id3 = item_id.astype(jnp.int32).reshape(n_workers, n_chunks, CHUNK)
    org_id3 = org_id.astype(jnp.int32).reshape(n_workers, n_chunks, CHUNK)
    ig, og = _sc_gather(item_id3, org_id3, item_table, org_table,
                        n_workers, n_chunks)
    return _tc_mlp(ig, og, W1, b1, W2, b2, W3, b3)
